# baseline (speedup 1.0000x reference)
import functools

import jax
import jax.numpy as jnp
from jax import lax
from jax.experimental import pallas as pl
from jax.experimental.pallas import tpu as pltpu
from jax.experimental.pallas import tpu_sc as plsc

N_U = 100000
N_I = 100000
D = 64
Q = 5
L = 2
TEMP = 0.2
LAMBDA_1 = 0.2
LAMBDA_2 = 1e-07
B = 1024

_TILE = 2000


_NNZ = 1200000
_EPT = _NNZ // 16
_BLK = 1024
_NBLK = -(-_EPT // _BLK)
_EPAD = 15 * _EPT + _NBLK * _BLK - _NNZ
_CAP = _BLK + 128
_NPASS = 4
_R = 12500
_ACC_ROWS = _R + 12
_ZROWS = _ACC_ROWS // 16
_WROWS = 782
_DUMMY = _R
_CH = 64
_DEPTH = 8


def _spmm_body(src_hbm, dst_hbm, vals_hbm, table_hbm, out_hbm,
               src_blk, dst_blk, vals_blk, sidx, didx, vals_c, didx2d,
               rows, zbuf, acc, gsem):
    c = lax.axis_index("c")
    s = lax.axis_index("s")
    lanes = lax.iota(jnp.int32, 16)
    tile_lo = s * _EPT
    tile_hi = tile_lo + _EPT
    zv = jnp.zeros((16,), jnp.float32)

    def zb(k, carry):
        for j in range(4):
            zbuf[k, pl.ds(j * 16, 16)] = zv
        return carry

    lax.fori_loop(0, 32, zb, 0)

    for p in range(_NPASS):
        base = (_NPASS * c + p) * _R

        zbase = s * _ZROWS

        def zc(j, carry):
            pltpu.sync_copy(zbuf, acc.at[pl.ds(zbase + j * 32, 32)])
            return carry

        lax.fori_loop(0, _ZROWS // 32, zc, 0)
        pltpu.sync_copy(zbuf.at[pl.ds(0, _ZROWS % 32)],
                        acc.at[pl.ds(zbase + (_ZROWS // 32) * 32, _ZROWS % 32)])
        plsc.subcore_barrier()

        def blk_body(b, carry):
            off = tile_lo + b * _BLK
            pltpu.sync_copy(src_hbm.at[pl.ds(off, _BLK)], src_blk)
            pltpu.sync_copy(dst_hbm.at[pl.ds(off, _BLK)], dst_blk)
            pltpu.sync_copy(vals_hbm.at[pl.ds(off, _BLK)], vals_blk)

            def pre(i, cc):
                sl = pl.ds(i * 16, 16)
                sidx[sl] = jnp.zeros((16,), jnp.int32)
                didx[sl] = jnp.full((16,), _DUMMY, jnp.int32)
                vals_c[sl] = zv
                return cc

            lax.fori_loop(0, _CAP // 16, pre, 0)

            def scan(i, ptr):
                sl = pl.ds(i * 16, 16)
                u = dst_blk[sl] - base
                g = off + i * 16 + lanes
                m = (u >= 0) & (u < _R) & (g < tile_hi)
                mi = jnp.where(m, 1, 0)
                cs = plsc.cumsum(mi)
                idx = (ptr + cs) - mi
                plsc.store_scatter(sidx, [idx], src_blk[sl], mask=m)
                plsc.store_scatter(didx, [idx], u, mask=m)
                plsc.store_scatter(vals_c, [idx], vals_blk[sl], mask=m)
                return ptr + cs[15]

            nc = lax.fori_loop(0, _BLK // 16, scan, 0)
            nch = (nc + _CH - 1) // _CH

            for d in range(_DEPTH):
                @pl.when(d < nch)
                def _():
                    pltpu.async_copy(table_hbm.at[sidx.at[pl.ds(d * _CH, _CH)]],
                                     rows.at[d], gsem)

            def chunk(k, cc):
                buf = k % _DEPTH
                for j in range(_CH // 16):
                    didx2d[k, pl.ds(j * 16, 16)] = didx[pl.ds(k * _CH + j * 16, 16)]
                pltpu.make_async_copy(
                    table_hbm.at[sidx.at[pl.ds(k * _CH, _CH)]],
                    rows.at[buf], gsem).wait()

                def scale(q, qq):
                    vv = vals_c[pl.ds(k * _CH + q * 16, 16)]
                    for t in range(16):
                        v = vv[t]
                        for j in range(4):
                            sl = pl.ds(j * 16, 16)
                            rows[buf, q * 16 + t, sl] = rows[buf, q * 16 + t, sl] * v
                    return qq

                lax.fori_loop(0, _CH // 16, scale, 0)
                pltpu.sync_copy(rows.at[buf], acc.at[didx2d.at[k]], add=True)

                @pl.when(k + _DEPTH < nch)
                def _():
                    pltpu.async_copy(
                        table_hbm.at[sidx.at[pl.ds((k + _DEPTH) * _CH, _CH)]],
                        rows.at[buf], gsem)
                return cc

            lax.fori_loop(0, nch, chunk, 0)
            return carry

        lax.fori_loop(0, _NBLK, blk_body, 0)
        plsc.subcore_barrier()

        wbase = s * _WROWS

        @pl.when(s < 15)
        def _():
            pltpu.sync_copy(acc.at[pl.ds(wbase, _WROWS)],
                            out_hbm.at[pl.ds(base + wbase, _WROWS)])

        @pl.when(s == 15)
        def _():
            pltpu.sync_copy(acc.at[pl.ds(15 * _WROWS, _R - 15 * _WROWS)],
                            out_hbm.at[pl.ds(base + 15 * _WROWS, _R - 15 * _WROWS)])

        plsc.subcore_barrier()


@functools.partial(
    pl.kernel,
    out_type=jax.ShapeDtypeStruct((N_U, D), jnp.float32),
    mesh=plsc.VectorSubcoreMesh(core_axis_name="c", subcore_axis_name="s"),
    compiler_params=pltpu.CompilerParams(needs_layout_passes=False,
                                         use_tc_tiling_on_sc=False),
    scratch_types=[
        pltpu.VMEM((_BLK,), jnp.int32),
        pltpu.VMEM((_BLK,), jnp.int32),
        pltpu.VMEM((_BLK,), jnp.float32),
        pltpu.VMEM((_CAP,), jnp.int32),
        pltpu.VMEM((_CAP,), jnp.int32),
        pltpu.VMEM((_CAP,), jnp.float32),
        pltpu.VMEM((_CAP // _CH, _CH), jnp.int32),
        pltpu.VMEM((_DEPTH, _CH, D), jnp.float32),
        pltpu.VMEM((32, D), jnp.float32),
        pltpu.VMEM_SHARED((_ACC_ROWS, D), jnp.float32),
        pltpu.SemaphoreType.DMA,
    ],
)
def _spmm_kernel(src_hbm, dst_hbm, vals_hbm, table_hbm, out_hbm, *scratch):
    _spmm_body(src_hbm, dst_hbm, vals_hbm, table_hbm, out_hbm, *scratch)


def _spmm(table, src, dst, vals):
    return _spmm_kernel(src, dst, vals, table)


def _flash_body(a_ref, b_ref, c_ref, g_ref, o_ref):
    i = pl.program_id(0)

    @pl.when(i == 0)
    def _():
        o_ref[...] = jnp.zeros_like(o_ref)

    e = a_ref[...] + b_ref[...] + c_ref[...]
    nsq = jnp.sum(e * e, axis=1)
    scale = lax.rsqrt(jnp.maximum(nsq, 1e-24)) * (1.0 / TEMP)
    logits = lax.dot_general(g_ref[...], e, (((1,), (1,)), ((), ())),
                             preferred_element_type=jnp.float32)
    s = jnp.exp(logits * scale[None, :])
    o_ref[...] += jnp.sum(s, axis=1, keepdims=True)


def _flash_sum(tab_a, tab_b, tab_c, g_rows):
    n = tab_a.shape[0]
    grid = (n // _TILE,)
    out = pl.pallas_call(
        _flash_body,
        grid=grid,
        in_specs=[
            pl.BlockSpec((_TILE, D), lambda i: (i, 0)),
            pl.BlockSpec((_TILE, D), lambda i: (i, 0)),
            pl.BlockSpec((_TILE, D), lambda i: (i, 0)),
            pl.BlockSpec((B, D), lambda i: (0, 0)),
        ],
        out_specs=pl.BlockSpec((B, 128), lambda i: (0, 0)),
        out_shape=jax.ShapeDtypeStruct((B, 128), jnp.float32),
    )(tab_a, tab_b, tab_c, g_rows)
    return out[:, 0]


def _l2n(x):
    return x / jnp.maximum(jnp.linalg.norm(x, axis=-1, keepdims=True), 1e-12)


def kernel(uids, iids, pos, neg, adj_rows, adj_cols, adj_vals,
           E_u_0, E_i_0, u_mul_s, v_mul_s, ut, vt):
    f32 = jnp.float32
    epad = _EPAD + (-_EPAD) % 8
    rowsP = jnp.pad(adj_rows.astype(jnp.int32), (0, epad), constant_values=N_U)
    colsP = jnp.pad(adj_cols.astype(jnp.int32), (0, epad), constant_values=N_I)
    valsP = jnp.pad(adj_vals, (0, epad))
    Z_u1 = _spmm(E_i_0, colsP, rowsP, valsP)
    Z_i1 = _spmm(E_u_0, rowsP, colsP, valsP)
    Z_u2 = _spmm(Z_i1, colsP, rowsP, valsP)
    Z_i2 = _spmm(Z_u1, rowsP, colsP, valsP)

    S_u = vt @ (E_i_0 + Z_i1)
    S_i = ut @ (E_u_0 + Z_u1)

    eu0_u, zu1_u, zu2_u = E_u_0[uids], Z_u1[uids], Z_u2[uids]
    ei0_i, zi1_i, zi2_i = E_i_0[iids], Z_i1[iids], Z_i2[iids]
    ei0_p, zi1_p, zi2_p = E_i_0[pos], Z_i1[pos], Z_i2[pos]
    ei0_n, zi1_n, zi2_n = E_i_0[neg], Z_i1[neg], Z_i2[neg]

    gu_rows = _l2n(eu0_u + u_mul_s[uids] @ S_u)
    gi_rows = _l2n(ei0_i + v_mul_s[iids] @ S_i)

    sum_u = _flash_sum(E_u_0, Z_u1, Z_u2, gu_rows)
    sum_i = _flash_sum(E_i_0, Z_i1, Z_i2, gi_rows)
    neg_score = jnp.log(sum_u + 1e-08).mean() + jnp.log(sum_i + 1e-08).mean()

    eu_rows = eu0_u + zu1_u + zu2_u
    ei_rows = ei0_i + zi1_i + zi2_i
    pos_score = (jnp.clip((gu_rows * _l2n(eu_rows)).sum(1) / TEMP, -5.0, 5.0).mean()
                 + jnp.clip((gi_rows * _l2n(ei_rows)).sum(1) / TEMP, -5.0, 5.0).mean())
    loss_s = -pos_score + neg_score

    pos_emb = ei0_p + zi1_p + zi2_p
    neg_emb = ei0_n + zi1_n + zi2_n
    pos_scores = (eu_rows * pos_emb).sum(-1)
    neg_scores = (eu_rows * neg_emb).sum(-1)
    loss_r = -jnp.log(jax.nn.sigmoid(pos_scores - neg_scores)).mean()

    loss_reg = (jnp.sum(E_u_0.astype(f32) ** 2)
                + jnp.sum(E_i_0.astype(f32) ** 2)) * LAMBDA_2
    loss = loss_r + loss_reg + LAMBDA_1 * loss_s
    return (loss, loss_r, LAMBDA_1 * loss_s)


# speedup vs baseline: 3.4185x; 3.4185x over previous
import functools

import jax
import jax.numpy as jnp
from jax import lax
from jax.experimental import pallas as pl
from jax.experimental.pallas import tpu as pltpu
from jax.experimental.pallas import tpu_sc as plsc

N_U = 100000
N_I = 100000
D = 64
Q = 5
L = 2
TEMP = 0.2
LAMBDA_1 = 0.2
LAMBDA_2 = 1e-07
B = 1024

_TILE = 2000


_NNZ = 1200000
_EPT = _NNZ // 16
_BLK = 1024
_NBLK = -(-_EPT // _BLK)
_EPAD = 15 * _EPT + _NBLK * _BLK - _NNZ
_CAP = 4096
_NPASS = 4
_R = 12500
_ACC_ROWS = _R + 12
_ZROWS = _ACC_ROWS // 16
_WROWS = 782
_DUMMY = _R
_CH = 64
_DEPTH = 12
_NRCH = _CAP // _CH


def _spmm_body(src_hbm, dst_hbm, vals_hbm, table_hbm, out_hbm,
               src_blk, dst_blk, vals_blk, sidx, didx, vals_c, didx2d,
               rows, zbuf, acc, gsem):
    c = lax.axis_index("c")
    s = lax.axis_index("s")
    lanes = lax.iota(jnp.int32, 16)
    tile_lo = s * _EPT
    tile_hi = tile_lo + _EPT
    zv = jnp.zeros((16,), jnp.float32)

    def zb(k, carry):
        for j in range(4):
            zbuf[k, pl.ds(j * 16, 16)] = zv
        return carry

    lax.fori_loop(0, 32, zb, 0)

    for p in range(_NPASS):
        base = (_NPASS * c + p) * _R

        zbase = s * _ZROWS

        def zc(j, carry):
            pltpu.sync_copy(zbuf, acc.at[pl.ds(zbase + j * 32, 32)])
            return carry

        lax.fori_loop(0, _ZROWS // 32, zc, 0)
        pltpu.sync_copy(zbuf.at[pl.ds(0, _ZROWS % 32)],
                        acc.at[pl.ds(zbase + (_ZROWS // 32) * 32, _ZROWS % 32)])
        plsc.subcore_barrier()

        def issue(i):
            start = (i % _NRCH) * _CH
            pltpu.async_copy(table_hbm.at[sidx.at[pl.ds(start, _CH)]],
                             rows.at[i % _DEPTH], gsem)

        def process(i):
            start = (i % _NRCH) * _CH
            pos = i % _NRCH
            buf = i % _DEPTH
            for j in range(_CH // 16):
                didx2d[pos, pl.ds(j * 16, 16)] = didx[pl.ds(start + j * 16, 16)]
            pltpu.make_async_copy(
                table_hbm.at[sidx.at[pl.ds(start, _CH)]],
                rows.at[buf], gsem).wait()

            def scale(q, qq):
                vv = vals_c[pl.ds(start + q * 16, 16)]
                for t in range(16):
                    v = vv[t]
                    for j in range(4):
                        sl = pl.ds(j * 16, 16)
                        rows[buf, q * 16 + t, sl] = rows[buf, q * 16 + t, sl] * v
                return qq

            lax.fori_loop(0, _CH // 16, scale, 0)
            pltpu.sync_copy(rows.at[buf], acc.at[didx2d.at[pos]], add=True)

        def pump(state, target):
            def cond(st):
                return st[0] < target

            def body(st):
                issued, done = st

                def full(d):
                    process(d)
                    return d + 1

                done = lax.cond(issued - done >= _DEPTH, full, lambda d: d, done)
                issue(issued)
                return (issued + 1, done)

            return lax.while_loop(cond, body, state)

        def drain_to(state, space_needed):
            def cond(st):
                return (wptr_ref[0] - st[1] * _CH) > (_CAP - space_needed)

            return state

        def blk_body(b, st):
            wptr, issued, done = st
            off = tile_lo + b * _BLK
            pltpu.sync_copy(src_hbm.at[pl.ds(off, _BLK)], src_blk)
            pltpu.sync_copy(dst_hbm.at[pl.ds(off, _BLK)], dst_blk)
            pltpu.sync_copy(vals_hbm.at[pl.ds(off, _BLK)], vals_blk)

            def scan(i, ptr):
                sl = pl.ds(i * 16, 16)
                u = dst_blk[sl] - base
                g = off + i * 16 + lanes
                m = (u >= 0) & (u < _R) & (g < tile_hi)
                mi = jnp.where(m, 1, 0)
                cs = plsc.cumsum(mi)
                idx = ((ptr + cs) - mi) & (_CAP - 1)
                plsc.store_scatter(sidx, [idx], src_blk[sl], mask=m)
                plsc.store_scatter(didx, [idx], u, mask=m)
                plsc.store_scatter(vals_c, [idx], vals_blk[sl], mask=m)
                return ptr + cs[15]

            wptr = lax.fori_loop(0, _BLK // 16, scan, wptr)
            issued, done = pump((issued, done), wptr // _CH)

            def cond2(st):
                return (wptr - st[1] * _CH) > (_CAP - _BLK)

            def body2(st):
                issued, done = st
                process(done)
                return (issued, done + 1)

            issued, done = lax.while_loop(cond2, body2, (issued, done))
            return (wptr, issued, done)

        wptr, issued, done = lax.fori_loop(0, _NBLK, blk_body, (0, 0, 0))

        pad = (-wptr) % _CH
        pidx = (wptr + lanes) & (_CAP - 1)
        pm = lanes < pad
        plsc.store_scatter(sidx, [pidx], jnp.zeros((16,), jnp.int32), mask=pm)
        plsc.store_scatter(didx, [pidx], jnp.full((16,), _DUMMY, jnp.int32), mask=pm)
        plsc.store_scatter(vals_c, [pidx], zv, mask=pm)
        pidx2 = (wptr + 16 + lanes) & (_CAP - 1)
        pm2 = (16 + lanes) < pad
        plsc.store_scatter(sidx, [pidx2], jnp.zeros((16,), jnp.int32), mask=pm2)
        plsc.store_scatter(didx, [pidx2], jnp.full((16,), _DUMMY, jnp.int32), mask=pm2)
        plsc.store_scatter(vals_c, [pidx2], zv, mask=pm2)
        pidx3 = (wptr + 32 + lanes) & (_CAP - 1)
        pm3 = (32 + lanes) < pad
        plsc.store_scatter(sidx, [pidx3], jnp.zeros((16,), jnp.int32), mask=pm3)
        plsc.store_scatter(didx, [pidx3], jnp.full((16,), _DUMMY, jnp.int32), mask=pm3)
        plsc.store_scatter(vals_c, [pidx3], zv, mask=pm3)
        pidx4 = (wptr + 48 + lanes) & (_CAP - 1)
        pm4 = (48 + lanes) < pad
        plsc.store_scatter(sidx, [pidx4], jnp.zeros((16,), jnp.int32), mask=pm4)
        plsc.store_scatter(didx, [pidx4], jnp.full((16,), _DUMMY, jnp.int32), mask=pm4)
        plsc.store_scatter(vals_c, [pidx4], zv, mask=pm4)
        wptr = wptr + pad

        issued, done = pump((issued, done), wptr // _CH)

        def cond3(st):
            return st[1] < issued

        def body3(st):
            i2, d2 = st
            process(d2)
            return (i2, d2 + 1)

        _, done = lax.while_loop(cond3, body3, (issued, done))

        plsc.subcore_barrier()

        wbase = s * _WROWS

        @pl.when(s < 15)
        def _():
            pltpu.sync_copy(acc.at[pl.ds(wbase, _WROWS)],
                            out_hbm.at[pl.ds(base + wbase, _WROWS)])

        @pl.when(s == 15)
        def _():
            pltpu.sync_copy(acc.at[pl.ds(15 * _WROWS, _R - 15 * _WROWS)],
                            out_hbm.at[pl.ds(base + 15 * _WROWS, _R - 15 * _WROWS)])

        plsc.subcore_barrier()


@functools.partial(
    pl.kernel,
    out_type=jax.ShapeDtypeStruct((N_U, D), jnp.float32),
    mesh=plsc.VectorSubcoreMesh(core_axis_name="c", subcore_axis_name="s"),
    compiler_params=pltpu.CompilerParams(needs_layout_passes=False,
                                         use_tc_tiling_on_sc=False),
    scratch_types=[
        pltpu.VMEM((_BLK,), jnp.int32),
        pltpu.VMEM((_BLK,), jnp.int32),
        pltpu.VMEM((_BLK,), jnp.float32),
        pltpu.VMEM((_CAP,), jnp.int32),
        pltpu.VMEM((_CAP,), jnp.int32),
        pltpu.VMEM((_CAP,), jnp.float32),
        pltpu.VMEM((_NRCH, _CH), jnp.int32),
        pltpu.VMEM((_DEPTH, _CH, D), jnp.float32),
        pltpu.VMEM((32, D), jnp.float32),
        pltpu.VMEM_SHARED((_ACC_ROWS, D), jnp.float32),
        pltpu.SemaphoreType.DMA,
    ],
)
def _spmm_kernel(src_hbm, dst_hbm, vals_hbm, table_hbm, out_hbm, *scratch):
    _spmm_body(src_hbm, dst_hbm, vals_hbm, table_hbm, out_hbm, *scratch)


def _spmm(table, src, dst, vals):
    return _spmm_kernel(src, dst, vals, table)


def _flash_body(a_ref, b_ref, c_ref, g_ref, o_ref):
    i = pl.program_id(0)

    @pl.when(i == 0)
    def _():
        o_ref[...] = jnp.zeros_like(o_ref)

    e = a_ref[...] + b_ref[...] + c_ref[...]
    nsq = jnp.sum(e * e, axis=1)
    scale = lax.rsqrt(jnp.maximum(nsq, 1e-24)) * (1.0 / TEMP)
    logits = lax.dot_general(g_ref[...], e, (((1,), (1,)), ((), ())),
                             preferred_element_type=jnp.float32)
    s = jnp.exp(logits * scale[None, :])
    o_ref[...] += jnp.sum(s, axis=1, keepdims=True)


def _flash_sum(tab_a, tab_b, tab_c, g_rows):
    n = tab_a.shape[0]
    grid = (n // _TILE,)
    out = pl.pallas_call(
        _flash_body,
        grid=grid,
        in_specs=[
            pl.BlockSpec((_TILE, D), lambda i: (i, 0)),
            pl.BlockSpec((_TILE, D), lambda i: (i, 0)),
            pl.BlockSpec((_TILE, D), lambda i: (i, 0)),
            pl.BlockSpec((B, D), lambda i: (0, 0)),
        ],
        out_specs=pl.BlockSpec((B, 128), lambda i: (0, 0)),
        out_shape=jax.ShapeDtypeStruct((B, 128), jnp.float32),
    )(tab_a, tab_b, tab_c, g_rows)
    return out[:, 0]


def _l2n(x):
    return x / jnp.maximum(jnp.linalg.norm(x, axis=-1, keepdims=True), 1e-12)


def kernel(uids, iids, pos, neg, adj_rows, adj_cols, adj_vals,
           E_u_0, E_i_0, u_mul_s, v_mul_s, ut, vt):
    f32 = jnp.float32
    epad = _EPAD + (-_EPAD) % 8
    rowsP = jnp.pad(adj_rows.astype(jnp.int32), (0, epad), constant_values=N_U)
    colsP = jnp.pad(adj_cols.astype(jnp.int32), (0, epad), constant_values=N_I)
    valsP = jnp.pad(adj_vals, (0, epad))
    Z_u1 = _spmm(E_i_0, colsP, rowsP, valsP)
    Z_i1 = _spmm(E_u_0, rowsP, colsP, valsP)
    Z_u2 = _spmm(Z_i1, colsP, rowsP, valsP)
    Z_i2 = _spmm(Z_u1, rowsP, colsP, valsP)

    S_u = vt @ (E_i_0 + Z_i1)
    S_i = ut @ (E_u_0 + Z_u1)

    eu0_u, zu1_u, zu2_u = E_u_0[uids], Z_u1[uids], Z_u2[uids]
    ei0_i, zi1_i, zi2_i = E_i_0[iids], Z_i1[iids], Z_i2[iids]
    ei0_p, zi1_p, zi2_p = E_i_0[pos], Z_i1[pos], Z_i2[pos]
    ei0_n, zi1_n, zi2_n = E_i_0[neg], Z_i1[neg], Z_i2[neg]

    gu_rows = _l2n(eu0_u + u_mul_s[uids] @ S_u)
    gi_rows = _l2n(ei0_i + v_mul_s[iids] @ S_i)

    sum_u = _flash_sum(E_u_0, Z_u1, Z_u2, gu_rows)
    sum_i = _flash_sum(E_i_0, Z_i1, Z_i2, gi_rows)
    neg_score = jnp.log(sum_u + 1e-08).mean() + jnp.log(sum_i + 1e-08).mean()

    eu_rows = eu0_u + zu1_u + zu2_u
    ei_rows = ei0_i + zi1_i + zi2_i
    pos_score = (jnp.clip((gu_rows * _l2n(eu_rows)).sum(1) / TEMP, -5.0, 5.0).mean()
                 + jnp.clip((gi_rows * _l2n(ei_rows)).sum(1) / TEMP, -5.0, 5.0).mean())
    loss_s = -pos_score + neg_score

    pos_emb = ei0_p + zi1_p + zi2_p
    neg_emb = ei0_n + zi1_n + zi2_n
    pos_scores = (eu_rows * pos_emb).sum(-1)
    neg_scores = (eu_rows * neg_emb).sum(-1)
    loss_r = -jnp.log(jax.nn.sigmoid(pos_scores - neg_scores)).mean()

    loss_reg = (jnp.sum(E_u_0.astype(f32) ** 2)
                + jnp.sum(E_i_0.astype(f32) ** 2)) * LAMBDA_2
    loss = loss_r + loss_reg + LAMBDA_1 * loss_s
    return (loss, loss_r, LAMBDA_1 * loss_s)


# speedup vs baseline: 4.9201x; 1.4393x over previous
import functools

import jax
import jax.numpy as jnp
from jax import lax
from jax.experimental import pallas as pl
from jax.experimental.pallas import tpu as pltpu
from jax.experimental.pallas import tpu_sc as plsc

N_U = 100000
N_I = 100000
D = 64
Q = 5
L = 2
TEMP = 0.2
LAMBDA_1 = 0.2
LAMBDA_2 = 1e-07
B = 1024

_TILE = 2000


_NNZ = 1200000
_EPT = _NNZ // 16
_BLK = 1024
_NBLK = -(-_EPT // _BLK)
_EPAD = 15 * _EPT + _NBLK * _BLK - _NNZ
_CAP = 4096
_NPASS = 4
_R = 12500
_ACC_ROWS = _R + 12
_ZROWS = _ACC_ROWS // 16
_WROWS = 782
_DUMMY = _R
_CH = 64
_DEPTH = 12
_NRCH = _CAP // _CH


def _spmm_body(src_hbm, dst_hbm, vals_hbm, table_hbm, out_hbm,
               src_blk, dst_blk, vals_blk, sidx, didx, vals_c, didx2d,
               rows, zbuf, acc, gsem, isem):
    c = lax.axis_index("c")
    s = lax.axis_index("s")
    lanes = lax.iota(jnp.int32, 16)
    tile_lo = s * _EPT
    tile_hi = tile_lo + _EPT
    zv = jnp.zeros((16,), jnp.float32)

    def zb(k, carry):
        for j in range(4):
            zbuf[k, pl.ds(j * 16, 16)] = zv
        return carry

    lax.fori_loop(0, 32, zb, 0)

    for p in range(_NPASS):
        base = (_NPASS * c + p) * _R

        zbase = s * _ZROWS

        def zc(j, carry):
            pltpu.sync_copy(zbuf, acc.at[pl.ds(zbase + j * 32, 32)])
            return carry

        lax.fori_loop(0, _ZROWS // 32, zc, 0)
        pltpu.sync_copy(zbuf.at[pl.ds(0, _ZROWS % 32)],
                        acc.at[pl.ds(zbase + (_ZROWS // 32) * 32, _ZROWS % 32)])
        plsc.subcore_barrier()

        def issue(i):
            start = (i % _NRCH) * _CH
            pltpu.async_copy(table_hbm.at[sidx.at[pl.ds(start, _CH)]],
                             rows.at[i % _DEPTH], gsem)

        def process(i):
            start = (i % _NRCH) * _CH
            pos = i % _NRCH
            buf = i % _DEPTH
            for j in range(_CH // 16):
                didx2d[pos, pl.ds(j * 16, 16)] = didx[pl.ds(start + j * 16, 16)]
            pltpu.make_async_copy(
                table_hbm.at[sidx.at[pl.ds(start, _CH)]],
                rows.at[buf], gsem).wait()

            def scale(q, qq):
                vv = vals_c[pl.ds(start + q * 16, 16)]
                for t in range(16):
                    v = vv[t]
                    for j in range(4):
                        sl = pl.ds(j * 16, 16)
                        rows[buf, q * 16 + t, sl] = rows[buf, q * 16 + t, sl] * v
                return qq

            lax.fori_loop(0, _CH // 16, scale, 0)
            pltpu.sync_copy(rows.at[buf], acc.at[didx2d.at[pos]], add=True)

        def pump(state, target):
            def cond(st):
                return st[0] < target

            def body(st):
                issued, done = st

                def full(d):
                    process(d)
                    return d + 1

                done = lax.cond(issued - done >= _DEPTH, full, lambda d: d, done)
                issue(issued)
                return (issued + 1, done)

            return lax.while_loop(cond, body, state)

        def drain_to(state, space_needed):
            def cond(st):
                return (wptr_ref[0] - st[1] * _CH) > (_CAP - space_needed)

            return state

        def load_blk(b):
            off = tile_lo + b * _BLK
            sel = b % 2
            pltpu.async_copy(src_hbm.at[pl.ds(off, _BLK)], src_blk.at[sel], isem)
            pltpu.async_copy(dst_hbm.at[pl.ds(off, _BLK)], dst_blk.at[sel], isem)
            pltpu.async_copy(vals_hbm.at[pl.ds(off, _BLK)], vals_blk.at[sel], isem)

        def wait_blk(b):
            off = tile_lo + b * _BLK
            sel = b % 2
            pltpu.make_async_copy(src_hbm.at[pl.ds(off, _BLK)], src_blk.at[sel], isem).wait()
            pltpu.make_async_copy(dst_hbm.at[pl.ds(off, _BLK)], dst_blk.at[sel], isem).wait()
            pltpu.make_async_copy(vals_hbm.at[pl.ds(off, _BLK)], vals_blk.at[sel], isem).wait()

        load_blk(0)

        def blk_body(b, st):
            wptr, issued, done = st
            off = tile_lo + b * _BLK
            sel = b % 2
            wait_blk(b)

            @pl.when(b + 1 < _NBLK)
            def _():
                load_blk(b + 1)

            def scan(i, ptr):
                for r in range(2):
                    sl = pl.ds((2 * i + r) * 16, 16)
                    u = dst_blk[sel, sl] - base
                    g = off + (2 * i + r) * 16 + lanes
                    m = (u >= 0) & (u < _R) & (g < tile_hi)
                    mi = jnp.where(m, 1, 0)
                    cs = plsc.cumsum(mi)
                    idx = ((ptr + cs) - mi) & (_CAP - 1)
                    plsc.store_scatter(sidx, [idx], src_blk[sel, sl], mask=m)
                    plsc.store_scatter(didx, [idx], u, mask=m)
                    plsc.store_scatter(vals_c, [idx], vals_blk[sel, sl], mask=m)
                    cnt = plsc.all_reduce_population_count(m)
                    ptr = ptr + cnt[0]
                return ptr

            wptr = lax.fori_loop(0, _BLK // 32, scan, wptr)
            issued, done = pump((issued, done), wptr // _CH)

            def cond2(st):
                return (wptr - st[1] * _CH) > (_CAP - _BLK)

            def body2(st):
                issued, done = st
                process(done)
                return (issued, done + 1)

            issued, done = lax.while_loop(cond2, body2, (issued, done))
            return (wptr, issued, done)

        wptr, issued, done = lax.fori_loop(0, _NBLK, blk_body, (0, 0, 0))

        pad = (-wptr) % _CH
        pidx = (wptr + lanes) & (_CAP - 1)
        pm = lanes < pad
        plsc.store_scatter(sidx, [pidx], jnp.zeros((16,), jnp.int32), mask=pm)
        plsc.store_scatter(didx, [pidx], jnp.full((16,), _DUMMY, jnp.int32), mask=pm)
        plsc.store_scatter(vals_c, [pidx], zv, mask=pm)
        pidx2 = (wptr + 16 + lanes) & (_CAP - 1)
        pm2 = (16 + lanes) < pad
        plsc.store_scatter(sidx, [pidx2], jnp.zeros((16,), jnp.int32), mask=pm2)
        plsc.store_scatter(didx, [pidx2], jnp.full((16,), _DUMMY, jnp.int32), mask=pm2)
        plsc.store_scatter(vals_c, [pidx2], zv, mask=pm2)
        pidx3 = (wptr + 32 + lanes) & (_CAP - 1)
        pm3 = (32 + lanes) < pad
        plsc.store_scatter(sidx, [pidx3], jnp.zeros((16,), jnp.int32), mask=pm3)
        plsc.store_scatter(didx, [pidx3], jnp.full((16,), _DUMMY, jnp.int32), mask=pm3)
        plsc.store_scatter(vals_c, [pidx3], zv, mask=pm3)
        pidx4 = (wptr + 48 + lanes) & (_CAP - 1)
        pm4 = (48 + lanes) < pad
        plsc.store_scatter(sidx, [pidx4], jnp.zeros((16,), jnp.int32), mask=pm4)
        plsc.store_scatter(didx, [pidx4], jnp.full((16,), _DUMMY, jnp.int32), mask=pm4)
        plsc.store_scatter(vals_c, [pidx4], zv, mask=pm4)
        wptr = wptr + pad

        issued, done = pump((issued, done), wptr // _CH)

        def cond3(st):
            return st[1] < issued

        def body3(st):
            i2, d2 = st
            process(d2)
            return (i2, d2 + 1)

        _, done = lax.while_loop(cond3, body3, (issued, done))

        plsc.subcore_barrier()

        wbase = s * _WROWS

        @pl.when(s < 15)
        def _():
            pltpu.sync_copy(acc.at[pl.ds(wbase, _WROWS)],
                            out_hbm.at[pl.ds(base + wbase, _WROWS)])

        @pl.when(s == 15)
        def _():
            pltpu.sync_copy(acc.at[pl.ds(15 * _WROWS, _R - 15 * _WROWS)],
                            out_hbm.at[pl.ds(base + 15 * _WROWS, _R - 15 * _WROWS)])

        plsc.subcore_barrier()


@functools.partial(
    pl.kernel,
    out_type=jax.ShapeDtypeStruct((N_U, D), jnp.float32),
    mesh=plsc.VectorSubcoreMesh(core_axis_name="c", subcore_axis_name="s"),
    compiler_params=pltpu.CompilerParams(needs_layout_passes=False,
                                         use_tc_tiling_on_sc=False),
    scratch_types=[
        pltpu.VMEM((2, _BLK), jnp.int32),
        pltpu.VMEM((2, _BLK), jnp.int32),
        pltpu.VMEM((2, _BLK), jnp.float32),
        pltpu.VMEM((_CAP,), jnp.int32),
        pltpu.VMEM((_CAP,), jnp.int32),
        pltpu.VMEM((_CAP,), jnp.float32),
        pltpu.VMEM((_NRCH, _CH), jnp.int32),
        pltpu.VMEM((_DEPTH, _CH, D), jnp.float32),
        pltpu.VMEM((32, D), jnp.float32),
        pltpu.VMEM_SHARED((_ACC_ROWS, D), jnp.float32),
        pltpu.SemaphoreType.DMA,
        pltpu.SemaphoreType.DMA,
    ],
)
def _spmm_kernel(src_hbm, dst_hbm, vals_hbm, table_hbm, out_hbm, *scratch):
    _spmm_body(src_hbm, dst_hbm, vals_hbm, table_hbm, out_hbm, *scratch)


def _spmm(table, src, dst, vals):
    return _spmm_kernel(src, dst, vals, table)


def _flash_body(a_ref, b_ref, c_ref, g_ref, o_ref):
    i = pl.program_id(0)

    @pl.when(i == 0)
    def _():
        o_ref[...] = jnp.zeros_like(o_ref)

    e = a_ref[...] + b_ref[...] + c_ref[...]
    nsq = jnp.sum(e * e, axis=1)
    scale = lax.rsqrt(jnp.maximum(nsq, 1e-24)) * (1.0 / TEMP)
    logits = lax.dot_general(g_ref[...], e, (((1,), (1,)), ((), ())),
                             preferred_element_type=jnp.float32)
    s = jnp.exp(logits * scale[None, :])
    o_ref[...] += jnp.sum(s, axis=1, keepdims=True)


def _flash_sum(tab_a, tab_b, tab_c, g_rows):
    n = tab_a.shape[0]
    grid = (n // _TILE,)
    out = pl.pallas_call(
        _flash_body,
        grid=grid,
        in_specs=[
            pl.BlockSpec((_TILE, D), lambda i: (i, 0)),
            pl.BlockSpec((_TILE, D), lambda i: (i, 0)),
            pl.BlockSpec((_TILE, D), lambda i: (i, 0)),
            pl.BlockSpec((B, D), lambda i: (0, 0)),
        ],
        out_specs=pl.BlockSpec((B, 128), lambda i: (0, 0)),
        out_shape=jax.ShapeDtypeStruct((B, 128), jnp.float32),
    )(tab_a, tab_b, tab_c, g_rows)
    return out[:, 0]


def _l2n(x):
    return x / jnp.maximum(jnp.linalg.norm(x, axis=-1, keepdims=True), 1e-12)


def kernel(uids, iids, pos, neg, adj_rows, adj_cols, adj_vals,
           E_u_0, E_i_0, u_mul_s, v_mul_s, ut, vt):
    f32 = jnp.float32
    epad = _EPAD + (-_EPAD) % 8
    rowsP = jnp.pad(adj_rows.astype(jnp.int32), (0, epad), constant_values=N_U)
    colsP = jnp.pad(adj_cols.astype(jnp.int32), (0, epad), constant_values=N_I)
    valsP = jnp.pad(adj_vals, (0, epad))
    Z_u1 = _spmm(E_i_0, colsP, rowsP, valsP)
    Z_i1 = _spmm(E_u_0, rowsP, colsP, valsP)
    Z_u2 = _spmm(Z_i1, colsP, rowsP, valsP)
    Z_i2 = _spmm(Z_u1, rowsP, colsP, valsP)

    S_u = vt @ (E_i_0 + Z_i1)
    S_i = ut @ (E_u_0 + Z_u1)

    eu0_u, zu1_u, zu2_u = E_u_0[uids], Z_u1[uids], Z_u2[uids]
    ei0_i, zi1_i, zi2_i = E_i_0[iids], Z_i1[iids], Z_i2[iids]
    ei0_p, zi1_p, zi2_p = E_i_0[pos], Z_i1[pos], Z_i2[pos]
    ei0_n, zi1_n, zi2_n = E_i_0[neg], Z_i1[neg], Z_i2[neg]

    gu_rows = _l2n(eu0_u + u_mul_s[uids] @ S_u)
    gi_rows = _l2n(ei0_i + v_mul_s[iids] @ S_i)

    sum_u = _flash_sum(E_u_0, Z_u1, Z_u2, gu_rows)
    sum_i = _flash_sum(E_i_0, Z_i1, Z_i2, gi_rows)
    neg_score = jnp.log(sum_u + 1e-08).mean() + jnp.log(sum_i + 1e-08).mean()

    eu_rows = eu0_u + zu1_u + zu2_u
    ei_rows = ei0_i + zi1_i + zi2_i
    pos_score = (jnp.clip((gu_rows * _l2n(eu_rows)).sum(1) / TEMP, -5.0, 5.0).mean()
                 + jnp.clip((gi_rows * _l2n(ei_rows)).sum(1) / TEMP, -5.0, 5.0).mean())
    loss_s = -pos_score + neg_score

    pos_emb = ei0_p + zi1_p + zi2_p
    neg_emb = ei0_n + zi1_n + zi2_n
    pos_scores = (eu_rows * pos_emb).sum(-1)
    neg_scores = (eu_rows * neg_emb).sum(-1)
    loss_r = -jnp.log(jax.nn.sigmoid(pos_scores - neg_scores)).mean()

    loss_reg = (jnp.sum(E_u_0.astype(f32) ** 2)
                + jnp.sum(E_i_0.astype(f32) ** 2)) * LAMBDA_2
    loss = loss_r + loss_reg + LAMBDA_1 * loss_s
    return (loss, loss_r, LAMBDA_1 * loss_s)


# speedup vs baseline: 4.9505x; 1.0062x over previous
import functools

import jax
import jax.numpy as jnp
from jax import lax
from jax.experimental import pallas as pl
from jax.experimental.pallas import tpu as pltpu
from jax.experimental.pallas import tpu_sc as plsc

N_U = 100000
N_I = 100000
D = 64
Q = 5
L = 2
TEMP = 0.2
LAMBDA_1 = 0.2
LAMBDA_2 = 1e-07
B = 1024

_TILE = 2000


_NNZ = 1200000
_EPT = _NNZ // 16
_BLK = 1024
_NBLK = -(-_EPT // _BLK)
_EPAD = 15 * _EPT + _NBLK * _BLK - _NNZ
_CAP = 4096
_NPASS = 4
_R = 12500
_ACC_ROWS = _R + 12
_ZROWS = _ACC_ROWS // 16
_WROWS = 782
_DUMMY = _R
_CH = 64
_DEPTH = 12
_NRCH = _CAP // _CH


def _spmm_body(src_hbm, dst_hbm, vals_hbm, table_hbm, out_hbm,
               src_blk, dst_blk, vals_blk, sidx, didx, vals_c, didx2d,
               rows, zbuf, acc, gsem, isem, ssem):
    c = lax.axis_index("c")
    s = lax.axis_index("s")
    lanes = lax.iota(jnp.int32, 16)
    tile_lo = s * _EPT
    tile_hi = tile_lo + _EPT
    zv = jnp.zeros((16,), jnp.float32)

    def zb(k, carry):
        for j in range(4):
            zbuf[k, pl.ds(j * 16, 16)] = zv
        return carry

    lax.fori_loop(0, 32, zb, 0)

    for p in range(_NPASS):
        base = (_NPASS * c + p) * _R

        zbase = s * _ZROWS

        def zc(j, carry):
            pltpu.sync_copy(zbuf, acc.at[pl.ds(zbase + j * 32, 32)])
            return carry

        lax.fori_loop(0, _ZROWS // 32, zc, 0)
        pltpu.sync_copy(zbuf.at[pl.ds(0, _ZROWS % 32)],
                        acc.at[pl.ds(zbase + (_ZROWS // 32) * 32, _ZROWS % 32)])
        plsc.subcore_barrier()

        def issue(i):
            start = (i % _NRCH) * _CH
            buf = i % _DEPTH

            @pl.when(i >= _DEPTH)
            def _():
                pltpu.make_async_copy(
                    rows.at[buf], acc.at[didx2d.at[(i - _DEPTH) % _NRCH]],
                    ssem).wait()

            pltpu.async_copy(table_hbm.at[sidx.at[pl.ds(start, _CH)]],
                             rows.at[buf], gsem)

        def process(i):
            start = (i % _NRCH) * _CH
            pos = i % _NRCH
            buf = i % _DEPTH
            for j in range(_CH // 16):
                didx2d[pos, pl.ds(j * 16, 16)] = didx[pl.ds(start + j * 16, 16)]
            pltpu.make_async_copy(
                table_hbm.at[sidx.at[pl.ds(start, _CH)]],
                rows.at[buf], gsem).wait()

            def scale(q, qq):
                vv = vals_c[pl.ds(start + q * 16, 16)]
                for t in range(16):
                    v = vv[t]
                    for j in range(4):
                        sl = pl.ds(j * 16, 16)
                        rows[buf, q * 16 + t, sl] = rows[buf, q * 16 + t, sl] * v
                return qq

            lax.fori_loop(0, _CH // 16, scale, 0)
            pltpu.async_copy(rows.at[buf], acc.at[didx2d.at[pos]], ssem,
                             add=True)

        def pump(state, target):
            def cond(st):
                return st[0] < target

            def body(st):
                issued, done = st

                def full(d):
                    process(d)
                    return d + 1

                done = lax.cond(issued - done >= _DEPTH, full, lambda d: d, done)
                issue(issued)
                return (issued + 1, done)

            return lax.while_loop(cond, body, state)

        def drain_to(state, space_needed):
            def cond(st):
                return (wptr_ref[0] - st[1] * _CH) > (_CAP - space_needed)

            return state

        def load_blk(b):
            off = tile_lo + b * _BLK
            sel = b % 2
            pltpu.async_copy(src_hbm.at[pl.ds(off, _BLK)], src_blk.at[sel], isem)
            pltpu.async_copy(dst_hbm.at[pl.ds(off, _BLK)], dst_blk.at[sel], isem)
            pltpu.async_copy(vals_hbm.at[pl.ds(off, _BLK)], vals_blk.at[sel], isem)

        def wait_blk(b):
            off = tile_lo + b * _BLK
            sel = b % 2
            pltpu.make_async_copy(src_hbm.at[pl.ds(off, _BLK)], src_blk.at[sel], isem).wait()
            pltpu.make_async_copy(dst_hbm.at[pl.ds(off, _BLK)], dst_blk.at[sel], isem).wait()
            pltpu.make_async_copy(vals_hbm.at[pl.ds(off, _BLK)], vals_blk.at[sel], isem).wait()

        load_blk(0)

        def blk_body(b, st):
            wptr, issued, done = st
            off = tile_lo + b * _BLK
            sel = b % 2
            wait_blk(b)

            @pl.when(b + 1 < _NBLK)
            def _():
                load_blk(b + 1)

            def scan(i, ptr):
                for r in range(2):
                    sl = pl.ds((2 * i + r) * 16, 16)
                    u = dst_blk[sel, sl] - base
                    g = off + (2 * i + r) * 16 + lanes
                    m = (u >= 0) & (u < _R) & (g < tile_hi)
                    mi = jnp.where(m, 1, 0)
                    cs = plsc.cumsum(mi)
                    idx = ((ptr + cs) - mi) & (_CAP - 1)
                    plsc.store_scatter(sidx, [idx], src_blk[sel, sl], mask=m)
                    plsc.store_scatter(didx, [idx], u, mask=m)
                    plsc.store_scatter(vals_c, [idx], vals_blk[sel, sl], mask=m)
                    cnt = plsc.all_reduce_population_count(m)
                    ptr = ptr + cnt[0]
                return ptr

            wptr = lax.fori_loop(0, _BLK // 32, scan, wptr)
            issued, done = pump((issued, done), wptr // _CH)

            def cond2(st):
                return (wptr - st[1] * _CH) > (_CAP - _BLK)

            def body2(st):
                issued, done = st
                process(done)
                return (issued, done + 1)

            issued, done = lax.while_loop(cond2, body2, (issued, done))
            return (wptr, issued, done)

        wptr, issued, done = lax.fori_loop(0, _NBLK, blk_body, (0, 0, 0))

        pad = (-wptr) % _CH
        pidx = (wptr + lanes) & (_CAP - 1)
        pm = lanes < pad
        plsc.store_scatter(sidx, [pidx], jnp.zeros((16,), jnp.int32), mask=pm)
        plsc.store_scatter(didx, [pidx], jnp.full((16,), _DUMMY, jnp.int32), mask=pm)
        plsc.store_scatter(vals_c, [pidx], zv, mask=pm)
        pidx2 = (wptr + 16 + lanes) & (_CAP - 1)
        pm2 = (16 + lanes) < pad
        plsc.store_scatter(sidx, [pidx2], jnp.zeros((16,), jnp.int32), mask=pm2)
        plsc.store_scatter(didx, [pidx2], jnp.full((16,), _DUMMY, jnp.int32), mask=pm2)
        plsc.store_scatter(vals_c, [pidx2], zv, mask=pm2)
        pidx3 = (wptr + 32 + lanes) & (_CAP - 1)
        pm3 = (32 + lanes) < pad
        plsc.store_scatter(sidx, [pidx3], jnp.zeros((16,), jnp.int32), mask=pm3)
        plsc.store_scatter(didx, [pidx3], jnp.full((16,), _DUMMY, jnp.int32), mask=pm3)
        plsc.store_scatter(vals_c, [pidx3], zv, mask=pm3)
        pidx4 = (wptr + 48 + lanes) & (_CAP - 1)
        pm4 = (48 + lanes) < pad
        plsc.store_scatter(sidx, [pidx4], jnp.zeros((16,), jnp.int32), mask=pm4)
        plsc.store_scatter(didx, [pidx4], jnp.full((16,), _DUMMY, jnp.int32), mask=pm4)
        plsc.store_scatter(vals_c, [pidx4], zv, mask=pm4)
        wptr = wptr + pad

        issued, done = pump((issued, done), wptr // _CH)

        def cond3(st):
            return st[1] < issued

        def body3(st):
            i2, d2 = st
            process(d2)
            return (i2, d2 + 1)

        _, done = lax.while_loop(cond3, body3, (issued, done))

        for d in range(_DEPTH):
            @pl.when(d < jnp.minimum(done, _DEPTH))
            def _():
                pltpu.make_async_copy(
                    rows.at[0], acc.at[didx2d.at[0]], ssem).wait()

        plsc.subcore_barrier()

        wbase = s * _WROWS

        @pl.when(s < 15)
        def _():
            pltpu.sync_copy(acc.at[pl.ds(wbase, _WROWS)],
                            out_hbm.at[pl.ds(base + wbase, _WROWS)])

        @pl.when(s == 15)
        def _():
            pltpu.sync_copy(acc.at[pl.ds(15 * _WROWS, _R - 15 * _WROWS)],
                            out_hbm.at[pl.ds(base + 15 * _WROWS, _R - 15 * _WROWS)])

        plsc.subcore_barrier()


@functools.partial(
    pl.kernel,
    out_type=jax.ShapeDtypeStruct((N_U, D), jnp.float32),
    mesh=plsc.VectorSubcoreMesh(core_axis_name="c", subcore_axis_name="s"),
    compiler_params=pltpu.CompilerParams(needs_layout_passes=False,
                                         use_tc_tiling_on_sc=False),
    scratch_types=[
        pltpu.VMEM((2, _BLK), jnp.int32),
        pltpu.VMEM((2, _BLK), jnp.int32),
        pltpu.VMEM((2, _BLK), jnp.float32),
        pltpu.VMEM((_CAP,), jnp.int32),
        pltpu.VMEM((_CAP,), jnp.int32),
        pltpu.VMEM((_CAP,), jnp.float32),
        pltpu.VMEM((_NRCH, _CH), jnp.int32),
        pltpu.VMEM((_DEPTH, _CH, D), jnp.float32),
        pltpu.VMEM((32, D), jnp.float32),
        pltpu.VMEM_SHARED((_ACC_ROWS, D), jnp.float32),
        pltpu.SemaphoreType.DMA,
        pltpu.SemaphoreType.DMA,
        pltpu.SemaphoreType.DMA,
    ],
)
def _spmm_kernel(src_hbm, dst_hbm, vals_hbm, table_hbm, out_hbm, *scratch):
    _spmm_body(src_hbm, dst_hbm, vals_hbm, table_hbm, out_hbm, *scratch)


def _spmm(table, src, dst, vals):
    return _spmm_kernel(src, dst, vals, table)


def _flash_body(a_ref, b_ref, c_ref, g_ref, o_ref):
    i = pl.program_id(0)

    @pl.when(i == 0)
    def _():
        o_ref[...] = jnp.zeros_like(o_ref)

    e = a_ref[...] + b_ref[...] + c_ref[...]
    nsq = jnp.sum(e * e, axis=1)
    scale = lax.rsqrt(jnp.maximum(nsq, 1e-24)) * (1.0 / TEMP)
    logits = lax.dot_general(g_ref[...], e, (((1,), (1,)), ((), ())),
                             preferred_element_type=jnp.float32)
    s = jnp.exp(logits * scale[None, :])
    o_ref[...] += jnp.sum(s, axis=1, keepdims=True)


def _flash_sum(tab_a, tab_b, tab_c, g_rows):
    n = tab_a.shape[0]
    grid = (n // _TILE,)
    out = pl.pallas_call(
        _flash_body,
        grid=grid,
        in_specs=[
            pl.BlockSpec((_TILE, D), lambda i: (i, 0)),
            pl.BlockSpec((_TILE, D), lambda i: (i, 0)),
            pl.BlockSpec((_TILE, D), lambda i: (i, 0)),
            pl.BlockSpec((B, D), lambda i: (0, 0)),
        ],
        out_specs=pl.BlockSpec((B, 128), lambda i: (0, 0)),
        out_shape=jax.ShapeDtypeStruct((B, 128), jnp.float32),
    )(tab_a, tab_b, tab_c, g_rows)
    return out[:, 0]


def _l2n(x):
    return x / jnp.maximum(jnp.linalg.norm(x, axis=-1, keepdims=True), 1e-12)


def kernel(uids, iids, pos, neg, adj_rows, adj_cols, adj_vals,
           E_u_0, E_i_0, u_mul_s, v_mul_s, ut, vt):
    f32 = jnp.float32
    epad = _EPAD + (-_EPAD) % 8
    rowsP = jnp.pad(adj_rows.astype(jnp.int32), (0, epad), constant_values=N_U)
    colsP = jnp.pad(adj_cols.astype(jnp.int32), (0, epad), constant_values=N_I)
    valsP = jnp.pad(adj_vals, (0, epad))
    Z_u1 = _spmm(E_i_0, colsP, rowsP, valsP)
    Z_i1 = _spmm(E_u_0, rowsP, colsP, valsP)
    Z_u2 = _spmm(Z_i1, colsP, rowsP, valsP)
    Z_i2 = _spmm(Z_u1, rowsP, colsP, valsP)

    S_u = vt @ (E_i_0 + Z_i1)
    S_i = ut @ (E_u_0 + Z_u1)

    eu0_u, zu1_u, zu2_u = E_u_0[uids], Z_u1[uids], Z_u2[uids]
    ei0_i, zi1_i, zi2_i = E_i_0[iids], Z_i1[iids], Z_i2[iids]
    ei0_p, zi1_p, zi2_p = E_i_0[pos], Z_i1[pos], Z_i2[pos]
    ei0_n, zi1_n, zi2_n = E_i_0[neg], Z_i1[neg], Z_i2[neg]

    gu_rows = _l2n(eu0_u + u_mul_s[uids] @ S_u)
    gi_rows = _l2n(ei0_i + v_mul_s[iids] @ S_i)

    sum_u = _flash_sum(E_u_0, Z_u1, Z_u2, gu_rows)
    sum_i = _flash_sum(E_i_0, Z_i1, Z_i2, gi_rows)
    neg_score = jnp.log(sum_u + 1e-08).mean() + jnp.log(sum_i + 1e-08).mean()

    eu_rows = eu0_u + zu1_u + zu2_u
    ei_rows = ei0_i + zi1_i + zi2_i
    pos_score = (jnp.clip((gu_rows * _l2n(eu_rows)).sum(1) / TEMP, -5.0, 5.0).mean()
                 + jnp.clip((gi_rows * _l2n(ei_rows)).sum(1) / TEMP, -5.0, 5.0).mean())
    loss_s = -pos_score + neg_score

    pos_emb = ei0_p + zi1_p + zi2_p
    neg_emb = ei0_n + zi1_n + zi2_n
    pos_scores = (eu_rows * pos_emb).sum(-1)
    neg_scores = (eu_rows * neg_emb).sum(-1)
    loss_r = -jnp.log(jax.nn.sigmoid(pos_scores - neg_scores)).mean()

    loss_reg = (jnp.sum(E_u_0.astype(f32) ** 2)
                + jnp.sum(E_i_0.astype(f32) ** 2)) * LAMBDA_2
    loss = loss_r + loss_reg + LAMBDA_1 * loss_s
    return (loss, loss_r, LAMBDA_1 * loss_s)


# speedup vs baseline: 9.3953x; 1.8979x over previous
import functools

import jax
import jax.numpy as jnp
from jax import lax
from jax.experimental import pallas as pl
from jax.experimental.pallas import tpu as pltpu
from jax.experimental.pallas import tpu_sc as plsc

N_U = 100000
N_I = 100000
D = 64
Q = 5
L = 2
TEMP = 0.2
LAMBDA_1 = 0.2
LAMBDA_2 = 1e-07
B = 1024

_TILE = 2000


_NNZ = 1200000
_EPT = _NNZ // 16
_BLK = 1024
_NBLK = -(-_EPT // _BLK)
_EPAD = 15 * _EPT + _NBLK * _BLK - _NNZ
_CAP = 2048
_NPASS = 2
_R = 25000
_ACC_ROWS = _R + 24
_ZROWS = _ACC_ROWS // 16
_WROWS = 1563
_DUMMY = _R
_CH = 64
_DEPTH = 3
_NRCH = _CAP // _CH


def _spmm_body(src_hbm, dst_hbm, vals_hbm, table_hbm, out_hbm,
               src_blk, dst_blk, vals_blk, sidx, didx, vals_c, didx2d,
               rows, zbuf, acc, gsem, isem, ssem):
    c = lax.axis_index("c")
    s = lax.axis_index("s")
    lanes = lax.iota(jnp.int32, 16)
    tile_lo = s * _EPT
    tile_hi = tile_lo + _EPT
    zv = jnp.zeros((16,), jnp.float32)

    def zb(k, carry):
        for j in range(4):
            zbuf[k, pl.ds(j * 16, 16)] = zv
        return carry

    lax.fori_loop(0, 32, zb, 0)

    for p in range(_NPASS):
        base = (_NPASS * c + p) * _R

        zbase = s * _ZROWS

        def zc(j, carry):
            pltpu.sync_copy(zbuf, acc.at[pl.ds(zbase + j * 32, 32)])
            return carry

        lax.fori_loop(0, _ZROWS // 32, zc, 0)
        pltpu.sync_copy(zbuf.at[pl.ds(0, _ZROWS % 32)],
                        acc.at[pl.ds(zbase + (_ZROWS // 32) * 32, _ZROWS % 32)])
        plsc.subcore_barrier()

        def issue(i):
            start = (i % _NRCH) * _CH
            buf = i % _DEPTH

            @pl.when(i >= _DEPTH)
            def _():
                pltpu.make_async_copy(
                    rows.at[buf], acc.at[didx2d.at[(i - _DEPTH) % _NRCH]],
                    ssem).wait()

            pltpu.async_copy(table_hbm.at[sidx.at[pl.ds(start, _CH)]],
                             rows.at[buf], gsem)

        def process(i):
            start = (i % _NRCH) * _CH
            pos = i % _NRCH
            buf = i % _DEPTH
            for j in range(_CH // 16):
                didx2d[pos, pl.ds(j * 16, 16)] = didx[pl.ds(start + j * 16, 16)]
            pltpu.make_async_copy(
                table_hbm.at[sidx.at[pl.ds(start, _CH)]],
                rows.at[buf], gsem).wait()

            def scale(q, qq):
                vv = vals_c[pl.ds(start + q * 16, 16)]
                for t in range(16):
                    v = vv[t]
                    for j in range(4):
                        sl = pl.ds(j * 16, 16)
                        rows[buf, q * 16 + t, sl] = rows[buf, q * 16 + t, sl] * v
                return qq

            lax.fori_loop(0, _CH // 16, scale, 0)
            pltpu.async_copy(rows.at[buf], acc.at[didx2d.at[pos]], ssem,
                             add=True)

        def pump(state, target):
            def cond(st):
                return st[0] < target

            def body(st):
                issued, done = st

                def full(d):
                    process(d)
                    return d + 1

                done = lax.cond(issued - done >= _DEPTH, full, lambda d: d, done)
                issue(issued)
                return (issued + 1, done)

            return lax.while_loop(cond, body, state)

        def drain_to(state, space_needed):
            def cond(st):
                return (wptr_ref[0] - st[1] * _CH) > (_CAP - space_needed)

            return state

        def load_blk(b):
            off = tile_lo + b * _BLK
            sel = b % 2
            pltpu.async_copy(src_hbm.at[pl.ds(off, _BLK)], src_blk.at[sel], isem)
            pltpu.async_copy(dst_hbm.at[pl.ds(off, _BLK)], dst_blk.at[sel], isem)
            pltpu.async_copy(vals_hbm.at[pl.ds(off, _BLK)], vals_blk.at[sel], isem)

        def wait_blk(b):
            off = tile_lo + b * _BLK
            sel = b % 2
            pltpu.make_async_copy(src_hbm.at[pl.ds(off, _BLK)], src_blk.at[sel], isem).wait()
            pltpu.make_async_copy(dst_hbm.at[pl.ds(off, _BLK)], dst_blk.at[sel], isem).wait()
            pltpu.make_async_copy(vals_hbm.at[pl.ds(off, _BLK)], vals_blk.at[sel], isem).wait()

        load_blk(0)

        def blk_body(b, st):
            wptr, issued, done = st
            off = tile_lo + b * _BLK
            sel = b % 2
            wait_blk(b)

            @pl.when(b + 1 < _NBLK)
            def _():
                load_blk(b + 1)

            def scan(i, ptr):
                for r in range(2):
                    sl = pl.ds((2 * i + r) * 16, 16)
                    u = dst_blk[sel, sl] - base
                    g = off + (2 * i + r) * 16 + lanes
                    m = (u >= 0) & (u < _R) & (g < tile_hi)
                    mi = jnp.where(m, 1, 0)
                    cs = plsc.cumsum(mi)
                    idx = ((ptr + cs) - mi) & (_CAP - 1)
                    plsc.store_scatter(sidx, [idx], src_blk[sel, sl], mask=m)
                    plsc.store_scatter(didx, [idx], u, mask=m)
                    plsc.store_scatter(vals_c, [idx], vals_blk[sel, sl], mask=m)
                    cnt = plsc.all_reduce_population_count(m)
                    ptr = ptr + cnt[0]
                return ptr

            wptr = lax.fori_loop(0, _BLK // 32, scan, wptr)
            issued, done = pump((issued, done), wptr // _CH)

            def cond2(st):
                return (wptr - st[1] * _CH) > (_CAP - _BLK)

            def body2(st):
                issued, done = st
                process(done)
                return (issued, done + 1)

            issued, done = lax.while_loop(cond2, body2, (issued, done))
            return (wptr, issued, done)

        wptr, issued, done = lax.fori_loop(0, _NBLK, blk_body, (0, 0, 0))

        pad = (-wptr) % _CH
        pidx = (wptr + lanes) & (_CAP - 1)
        pm = lanes < pad
        plsc.store_scatter(sidx, [pidx], jnp.zeros((16,), jnp.int32), mask=pm)
        plsc.store_scatter(didx, [pidx], jnp.full((16,), _DUMMY, jnp.int32), mask=pm)
        plsc.store_scatter(vals_c, [pidx], zv, mask=pm)
        pidx2 = (wptr + 16 + lanes) & (_CAP - 1)
        pm2 = (16 + lanes) < pad
        plsc.store_scatter(sidx, [pidx2], jnp.zeros((16,), jnp.int32), mask=pm2)
        plsc.store_scatter(didx, [pidx2], jnp.full((16,), _DUMMY, jnp.int32), mask=pm2)
        plsc.store_scatter(vals_c, [pidx2], zv, mask=pm2)
        pidx3 = (wptr + 32 + lanes) & (_CAP - 1)
        pm3 = (32 + lanes) < pad
        plsc.store_scatter(sidx, [pidx3], jnp.zeros((16,), jnp.int32), mask=pm3)
        plsc.store_scatter(didx, [pidx3], jnp.full((16,), _DUMMY, jnp.int32), mask=pm3)
        plsc.store_scatter(vals_c, [pidx3], zv, mask=pm3)
        pidx4 = (wptr + 48 + lanes) & (_CAP - 1)
        pm4 = (48 + lanes) < pad
        plsc.store_scatter(sidx, [pidx4], jnp.zeros((16,), jnp.int32), mask=pm4)
        plsc.store_scatter(didx, [pidx4], jnp.full((16,), _DUMMY, jnp.int32), mask=pm4)
        plsc.store_scatter(vals_c, [pidx4], zv, mask=pm4)
        wptr = wptr + pad

        issued, done = pump((issued, done), wptr // _CH)

        def cond3(st):
            return st[1] < issued

        def body3(st):
            i2, d2 = st
            process(d2)
            return (i2, d2 + 1)

        _, done = lax.while_loop(cond3, body3, (issued, done))

        for d in range(_DEPTH):
            @pl.when(d < jnp.minimum(done, _DEPTH))
            def _():
                pltpu.make_async_copy(
                    rows.at[0], acc.at[didx2d.at[0]], ssem).wait()

        plsc.subcore_barrier()

        wbase = s * _WROWS

        @pl.when(s < 15)
        def _():
            pltpu.sync_copy(acc.at[pl.ds(wbase, _WROWS)],
                            out_hbm.at[pl.ds(base + wbase, _WROWS)])

        @pl.when(s == 15)
        def _():
            pltpu.sync_copy(acc.at[pl.ds(15 * _WROWS, _R - 15 * _WROWS)],
                            out_hbm.at[pl.ds(base + 15 * _WROWS, _R - 15 * _WROWS)])

        plsc.subcore_barrier()


@functools.partial(
    pl.kernel,
    out_type=jax.ShapeDtypeStruct((N_U, D), jnp.float32),
    mesh=plsc.VectorSubcoreMesh(core_axis_name="c", subcore_axis_name="s"),
    compiler_params=pltpu.CompilerParams(needs_layout_passes=False,
                                         use_tc_tiling_on_sc=False),
    scratch_types=[
        pltpu.VMEM((2, _BLK), jnp.int32),
        pltpu.VMEM((2, _BLK), jnp.int32),
        pltpu.VMEM((2, _BLK), jnp.float32),
        pltpu.VMEM((_CAP,), jnp.int32),
        pltpu.VMEM((_CAP,), jnp.int32),
        pltpu.VMEM((_CAP,), jnp.float32),
        pltpu.VMEM((_NRCH, _CH), jnp.int32),
        pltpu.VMEM((_DEPTH, _CH, D), jnp.float32),
        pltpu.VMEM((32, D), jnp.float32),
        pltpu.VMEM_SHARED((_ACC_ROWS, D), jnp.float32),
        pltpu.SemaphoreType.DMA,
        pltpu.SemaphoreType.DMA,
        pltpu.SemaphoreType.DMA,
    ],
)
def _spmm_kernel(src_hbm, dst_hbm, vals_hbm, table_hbm, out_hbm, *scratch):
    _spmm_body(src_hbm, dst_hbm, vals_hbm, table_hbm, out_hbm, *scratch)


def _spmm(table, src, dst, vals):
    return _spmm_kernel(src, dst, vals, table)


def _flash_body(a_ref, b_ref, c_ref, g_ref, o_ref):
    i = pl.program_id(0)

    @pl.when(i == 0)
    def _():
        o_ref[...] = jnp.zeros_like(o_ref)

    e = a_ref[...] + b_ref[...] + c_ref[...]
    nsq = jnp.sum(e * e, axis=1)
    scale = lax.rsqrt(jnp.maximum(nsq, 1e-24)) * (1.0 / TEMP)
    logits = lax.dot_general(g_ref[...], e, (((1,), (1,)), ((), ())),
                             preferred_element_type=jnp.float32)
    s = jnp.exp(logits * scale[None, :])
    o_ref[...] += jnp.sum(s, axis=1, keepdims=True)


def _flash_sum(tab_a, tab_b, tab_c, g_rows):
    n = tab_a.shape[0]
    grid = (n // _TILE,)
    out = pl.pallas_call(
        _flash_body,
        grid=grid,
        in_specs=[
            pl.BlockSpec((_TILE, D), lambda i: (i, 0)),
            pl.BlockSpec((_TILE, D), lambda i: (i, 0)),
            pl.BlockSpec((_TILE, D), lambda i: (i, 0)),
            pl.BlockSpec((B, D), lambda i: (0, 0)),
        ],
        out_specs=pl.BlockSpec((B, 128), lambda i: (0, 0)),
        out_shape=jax.ShapeDtypeStruct((B, 128), jnp.float32),
    )(tab_a, tab_b, tab_c, g_rows)
    return out[:, 0]


def _l2n(x):
    return x / jnp.maximum(jnp.linalg.norm(x, axis=-1, keepdims=True), 1e-12)


def kernel(uids, iids, pos, neg, adj_rows, adj_cols, adj_vals,
           E_u_0, E_i_0, u_mul_s, v_mul_s, ut, vt):
    f32 = jnp.float32
    epad = _EPAD + (-_EPAD) % 8
    rowsP = jnp.pad(adj_rows.astype(jnp.int32), (0, epad), constant_values=N_U)
    colsP = jnp.pad(adj_cols.astype(jnp.int32), (0, epad), constant_values=N_I)
    valsP = jnp.pad(adj_vals, (0, epad))
    Z_u1 = _spmm(E_i_0, colsP, rowsP, valsP)
    Z_i1 = _spmm(E_u_0, rowsP, colsP, valsP)
    Z_u2 = _spmm(Z_i1, colsP, rowsP, valsP)
    Z_i2 = _spmm(Z_u1, rowsP, colsP, valsP)

    S_u = vt @ (E_i_0 + Z_i1)
    S_i = ut @ (E_u_0 + Z_u1)

    eu0_u, zu1_u, zu2_u = E_u_0[uids], Z_u1[uids], Z_u2[uids]
    ei0_i, zi1_i, zi2_i = E_i_0[iids], Z_i1[iids], Z_i2[iids]
    ei0_p, zi1_p, zi2_p = E_i_0[pos], Z_i1[pos], Z_i2[pos]
    ei0_n, zi1_n, zi2_n = E_i_0[neg], Z_i1[neg], Z_i2[neg]

    gu_rows = _l2n(eu0_u + u_mul_s[uids] @ S_u)
    gi_rows = _l2n(ei0_i + v_mul_s[iids] @ S_i)

    sum_u = _flash_sum(E_u_0, Z_u1, Z_u2, gu_rows)
    sum_i = _flash_sum(E_i_0, Z_i1, Z_i2, gi_rows)
    neg_score = jnp.log(sum_u + 1e-08).mean() + jnp.log(sum_i + 1e-08).mean()

    eu_rows = eu0_u + zu1_u + zu2_u
    ei_rows = ei0_i + zi1_i + zi2_i
    pos_score = (jnp.clip((gu_rows * _l2n(eu_rows)).sum(1) / TEMP, -5.0, 5.0).mean()
                 + jnp.clip((gi_rows * _l2n(ei_rows)).sum(1) / TEMP, -5.0, 5.0).mean())
    loss_s = -pos_score + neg_score

    pos_emb = ei0_p + zi1_p + zi2_p
    neg_emb = ei0_n + zi1_n + zi2_n
    pos_scores = (eu_rows * pos_emb).sum(-1)
    neg_scores = (eu_rows * neg_emb).sum(-1)
    loss_r = -jnp.log(jax.nn.sigmoid(pos_scores - neg_scores)).mean()

    loss_reg = (jnp.sum(E_u_0.astype(f32) ** 2)
                + jnp.sum(E_i_0.astype(f32) ** 2)) * LAMBDA_2
    loss = loss_r + loss_reg + LAMBDA_1 * loss_s
    return (loss, loss_r, LAMBDA_1 * loss_s)


# speedup vs baseline: 9.4004x; 1.0005x over previous
import functools

import jax
import jax.numpy as jnp
from jax import lax
from jax.experimental import pallas as pl
from jax.experimental.pallas import tpu as pltpu
from jax.experimental.pallas import tpu_sc as plsc

N_U = 100000
N_I = 100000
D = 64
Q = 5
L = 2
TEMP = 0.2
LAMBDA_1 = 0.2
LAMBDA_2 = 1e-07
B = 1024

_TILE = 2000


_NNZ = 1200000
_EPT = _NNZ // 16
_BLK = 1024
_NBLK = -(-_EPT // _BLK)
_EPAD = 15 * _EPT + _NBLK * _BLK - _NNZ
_CAP = 2048
_NPASS = 2
_R = 25000
_ACC_ROWS = _R + 24
_ZROWS = _ACC_ROWS // 16
_WROWS = 1563
_DUMMY = _R
_CH = 64
_DEPTH = 3
_NRCH = _CAP // _CH


def _spmm_body(src_hbm, dst_hbm, vals_hbm, table_hbm, out_hbm,
               src_blk, dst_blk, vals_blk, sidx, didx, vals_c, didx2d,
               rows, zbuf, acc, gsem, isem, ssem):
    c = lax.axis_index("c")
    s = lax.axis_index("s")
    lanes = lax.iota(jnp.int32, 16)
    tile_lo = s * _EPT
    tile_hi = tile_lo + _EPT
    zv = jnp.zeros((16,), jnp.float32)

    def zb(k, carry):
        for j in range(4):
            zbuf[k, pl.ds(j * 16, 16)] = zv
        return carry

    lax.fori_loop(0, 32, zb, 0)

    for p in range(_NPASS):
        base = (_NPASS * c + p) * _R

        zbase = s * _ZROWS

        def zc(j, carry):
            pltpu.sync_copy(zbuf, acc.at[pl.ds(zbase + j * 32, 32)])
            return carry

        lax.fori_loop(0, _ZROWS // 32, zc, 0)
        pltpu.sync_copy(zbuf.at[pl.ds(0, _ZROWS % 32)],
                        acc.at[pl.ds(zbase + (_ZROWS // 32) * 32, _ZROWS % 32)])
        plsc.subcore_barrier()

        def issue(i):
            start = (i % _NRCH) * _CH
            buf = i % _DEPTH

            @pl.when(i >= _DEPTH)
            def _():
                pltpu.make_async_copy(
                    rows.at[buf], acc.at[didx2d.at[(i - _DEPTH) % _NRCH]],
                    ssem).wait()

            pltpu.async_copy(table_hbm.at[sidx.at[pl.ds(start, _CH)]],
                             rows.at[buf], gsem)

        def process(i):
            start = (i % _NRCH) * _CH
            pos = i % _NRCH
            buf = i % _DEPTH
            for j in range(_CH // 16):
                didx2d[pos, pl.ds(j * 16, 16)] = didx[pl.ds(start + j * 16, 16)]
            pltpu.make_async_copy(
                table_hbm.at[sidx.at[pl.ds(start, _CH)]],
                rows.at[buf], gsem).wait()

            def scale(q, qq):
                vv = vals_c[pl.ds(start + q * 16, 16)]
                for t in range(16):
                    v = vv[t]
                    for j in range(4):
                        sl = pl.ds(j * 16, 16)
                        rows[buf, q * 16 + t, sl] = rows[buf, q * 16 + t, sl] * v
                return qq

            lax.fori_loop(0, _CH // 16, scale, 0)
            pltpu.async_copy(rows.at[buf], acc.at[didx2d.at[pos]], ssem,
                             add=True)

        def pump(state, target):
            def cond(st):
                return st[0] < target

            def body(st):
                issued, done = st

                def full(d):
                    process(d)
                    return d + 1

                done = lax.cond(issued - done >= _DEPTH, full, lambda d: d, done)
                issue(issued)
                return (issued + 1, done)

            return lax.while_loop(cond, body, state)

        def load_blk(b):
            off = tile_lo + b * _BLK
            sel = b % 2
            pltpu.async_copy(src_hbm.at[pl.ds(off, _BLK)], src_blk.at[sel], isem)
            pltpu.async_copy(dst_hbm.at[pl.ds(off, _BLK)], dst_blk.at[sel], isem)
            pltpu.async_copy(vals_hbm.at[pl.ds(off, _BLK)], vals_blk.at[sel], isem)

        def wait_blk(b):
            off = tile_lo + b * _BLK
            sel = b % 2
            pltpu.make_async_copy(src_hbm.at[pl.ds(off, _BLK)], src_blk.at[sel], isem).wait()
            pltpu.make_async_copy(dst_hbm.at[pl.ds(off, _BLK)], dst_blk.at[sel], isem).wait()
            pltpu.make_async_copy(vals_hbm.at[pl.ds(off, _BLK)], vals_blk.at[sel], isem).wait()

        load_blk(0)

        def blk_body(b, st):
            wptr, issued, done = st
            off = tile_lo + b * _BLK
            sel = b % 2
            wait_blk(b)

            @pl.when(b + 1 < _NBLK)
            def _():
                load_blk(b + 1)

            def scan(i, ptr):
                for r in range(2):
                    sl = pl.ds((2 * i + r) * 16, 16)
                    u = dst_blk[sel, sl] - base
                    g = off + (2 * i + r) * 16 + lanes
                    m = (u >= 0) & (u < _R) & (g < tile_hi)
                    mi = jnp.where(m, 1, 0)
                    cs = plsc.cumsum(mi)
                    idx = ((ptr + cs) - mi) & (_CAP - 1)
                    plsc.store_scatter(sidx, [idx], src_blk[sel, sl], mask=m)
                    plsc.store_scatter(didx, [idx], u, mask=m)
                    plsc.store_scatter(vals_c, [idx], vals_blk[sel, sl], mask=m)
                    cnt = plsc.all_reduce_population_count(m)
                    ptr = ptr + cnt[0]
                return ptr

            wptr = lax.fori_loop(0, _BLK // 32, scan, wptr)
            issued, done = pump((issued, done), wptr // _CH)

            def cond2(st):
                return (wptr - st[1] * _CH) > (_CAP - _BLK)

            def body2(st):
                issued, done = st
                process(done)
                return (issued, done + 1)

            issued, done = lax.while_loop(cond2, body2, (issued, done))
            return (wptr, issued, done)

        wptr, issued, done = lax.fori_loop(0, _NBLK, blk_body, (0, 0, 0))

        pad = (-wptr) % _CH
        pidx = (wptr + lanes) & (_CAP - 1)
        pm = lanes < pad
        plsc.store_scatter(sidx, [pidx], jnp.zeros((16,), jnp.int32), mask=pm)
        plsc.store_scatter(didx, [pidx], jnp.full((16,), _DUMMY, jnp.int32), mask=pm)
        plsc.store_scatter(vals_c, [pidx], zv, mask=pm)
        pidx2 = (wptr + 16 + lanes) & (_CAP - 1)
        pm2 = (16 + lanes) < pad
        plsc.store_scatter(sidx, [pidx2], jnp.zeros((16,), jnp.int32), mask=pm2)
        plsc.store_scatter(didx, [pidx2], jnp.full((16,), _DUMMY, jnp.int32), mask=pm2)
        plsc.store_scatter(vals_c, [pidx2], zv, mask=pm2)
        pidx3 = (wptr + 32 + lanes) & (_CAP - 1)
        pm3 = (32 + lanes) < pad
        plsc.store_scatter(sidx, [pidx3], jnp.zeros((16,), jnp.int32), mask=pm3)
        plsc.store_scatter(didx, [pidx3], jnp.full((16,), _DUMMY, jnp.int32), mask=pm3)
        plsc.store_scatter(vals_c, [pidx3], zv, mask=pm3)
        pidx4 = (wptr + 48 + lanes) & (_CAP - 1)
        pm4 = (48 + lanes) < pad
        plsc.store_scatter(sidx, [pidx4], jnp.zeros((16,), jnp.int32), mask=pm4)
        plsc.store_scatter(didx, [pidx4], jnp.full((16,), _DUMMY, jnp.int32), mask=pm4)
        plsc.store_scatter(vals_c, [pidx4], zv, mask=pm4)
        wptr = wptr + pad

        issued, done = pump((issued, done), wptr // _CH)

        def cond3(st):
            return st[1] < issued

        def body3(st):
            i2, d2 = st
            process(d2)
            return (i2, d2 + 1)

        _, done = lax.while_loop(cond3, body3, (issued, done))

        for d in range(_DEPTH):
            @pl.when(d < jnp.minimum(done, _DEPTH))
            def _():
                pltpu.make_async_copy(
                    rows.at[0], acc.at[didx2d.at[0]], ssem).wait()

        plsc.subcore_barrier()

        wbase = s * _WROWS

        @pl.when(s < 15)
        def _():
            pltpu.sync_copy(acc.at[pl.ds(wbase, _WROWS)],
                            out_hbm.at[pl.ds(base + wbase, _WROWS)])

        @pl.when(s == 15)
        def _():
            pltpu.sync_copy(acc.at[pl.ds(15 * _WROWS, _R - 15 * _WROWS)],
                            out_hbm.at[pl.ds(base + 15 * _WROWS, _R - 15 * _WROWS)])

        plsc.subcore_barrier()


@functools.partial(
    pl.kernel,
    out_type=jax.ShapeDtypeStruct((N_U, D), jnp.float32),
    mesh=plsc.VectorSubcoreMesh(core_axis_name="c", subcore_axis_name="s"),
    compiler_params=pltpu.CompilerParams(needs_layout_passes=False,
                                         use_tc_tiling_on_sc=False),
    scratch_types=[
        pltpu.VMEM((2, _BLK), jnp.int32),
        pltpu.VMEM((2, _BLK), jnp.int32),
        pltpu.VMEM((2, _BLK), jnp.float32),
        pltpu.VMEM((_CAP,), jnp.int32),
        pltpu.VMEM((_CAP,), jnp.int32),
        pltpu.VMEM((_CAP,), jnp.float32),
        pltpu.VMEM((_NRCH, _CH), jnp.int32),
        pltpu.VMEM((_DEPTH, _CH, D), jnp.float32),
        pltpu.VMEM((32, D), jnp.float32),
        pltpu.VMEM_SHARED((_ACC_ROWS, D), jnp.float32),
        pltpu.SemaphoreType.DMA,
        pltpu.SemaphoreType.DMA,
        pltpu.SemaphoreType.DMA,
    ],
)
def _spmm_kernel(src_hbm, dst_hbm, vals_hbm, table_hbm, out_hbm, *scratch):
    _spmm_body(src_hbm, dst_hbm, vals_hbm, table_hbm, out_hbm, *scratch)


def _spmm(table, src, dst, vals):
    return _spmm_kernel(src, dst, vals, table)


def _flash_body(a_ref, b_ref, c_ref, g_ref, o_ref):
    i = pl.program_id(0)

    @pl.when(i == 0)
    def _():
        o_ref[...] = jnp.zeros_like(o_ref)

    e = a_ref[...] + b_ref[...] + c_ref[...]
    nsq = jnp.sum(e * e, axis=1)
    scale = lax.rsqrt(jnp.maximum(nsq, 1e-24)) * (1.0 / TEMP)
    logits = lax.dot_general(g_ref[...], e, (((1,), (1,)), ((), ())),
                             preferred_element_type=jnp.float32)
    s = jnp.exp(logits * scale[None, :])
    o_ref[...] += jnp.sum(s, axis=1, keepdims=True)


def _flash_sum(tab_a, tab_b, tab_c, g_rows):
    n = tab_a.shape[0]
    grid = (n // _TILE,)
    out = pl.pallas_call(
        _flash_body,
        grid=grid,
        in_specs=[
            pl.BlockSpec((_TILE, D), lambda i: (i, 0)),
            pl.BlockSpec((_TILE, D), lambda i: (i, 0)),
            pl.BlockSpec((_TILE, D), lambda i: (i, 0)),
            pl.BlockSpec((B, D), lambda i: (0, 0)),
        ],
        out_specs=pl.BlockSpec((B, 128), lambda i: (0, 0)),
        out_shape=jax.ShapeDtypeStruct((B, 128), jnp.float32),
    )(tab_a, tab_b, tab_c, g_rows)
    return out[:, 0]


def _l2n(x):
    return x / jnp.maximum(jnp.linalg.norm(x, axis=-1, keepdims=True), 1e-12)


def kernel(uids, iids, pos, neg, adj_rows, adj_cols, adj_vals,
           E_u_0, E_i_0, u_mul_s, v_mul_s, ut, vt):
    f32 = jnp.float32
    epad = _EPAD + (-_EPAD) % 8
    rowsP = jnp.pad(adj_rows.astype(jnp.int32), (0, epad), constant_values=N_U)
    colsP = jnp.pad(adj_cols.astype(jnp.int32), (0, epad), constant_values=N_I)
    valsP = jnp.pad(adj_vals, (0, epad))
    Z_u1 = _spmm(E_i_0, colsP, rowsP, valsP)
    Z_i1 = _spmm(E_u_0, rowsP, colsP, valsP)
    Z_u2 = _spmm(Z_i1, colsP, rowsP, valsP)
    Z_i2 = _spmm(Z_u1, rowsP, colsP, valsP)

    S_u = vt @ (E_i_0 + Z_i1)
    S_i = ut @ (E_u_0 + Z_u1)

    eu0_u, zu1_u, zu2_u = E_u_0[uids], Z_u1[uids], Z_u2[uids]
    ei0_i, zi1_i, zi2_i = E_i_0[iids], Z_i1[iids], Z_i2[iids]
    ei0_p, zi1_p, zi2_p = E_i_0[pos], Z_i1[pos], Z_i2[pos]
    ei0_n, zi1_n, zi2_n = E_i_0[neg], Z_i1[neg], Z_i2[neg]

    gu_rows = _l2n(eu0_u + u_mul_s[uids] @ S_u)
    gi_rows = _l2n(ei0_i + v_mul_s[iids] @ S_i)

    sum_u = _flash_sum(E_u_0, Z_u1, Z_u2, gu_rows)
    sum_i = _flash_sum(E_i_0, Z_i1, Z_i2, gi_rows)
    neg_score = jnp.log(sum_u + 1e-08).mean() + jnp.log(sum_i + 1e-08).mean()

    eu_rows = eu0_u + zu1_u + zu2_u
    ei_rows = ei0_i + zi1_i + zi2_i
    pos_score = (jnp.clip((gu_rows * _l2n(eu_rows)).sum(1) / TEMP, -5.0, 5.0).mean()
                 + jnp.clip((gi_rows * _l2n(ei_rows)).sum(1) / TEMP, -5.0, 5.0).mean())
    loss_s = -pos_score + neg_score

    pos_emb = ei0_p + zi1_p + zi2_p
    neg_emb = ei0_n + zi1_n + zi2_n
    pos_scores = (eu_rows * pos_emb).sum(-1)
    neg_scores = (eu_rows * neg_emb).sum(-1)
    loss_r = -jnp.log(jax.nn.sigmoid(pos_scores - neg_scores)).mean()

    loss_reg = (jnp.sum(E_u_0.astype(f32) ** 2)
                + jnp.sum(E_i_0.astype(f32) ** 2)) * LAMBDA_2
    loss = loss_r + loss_reg + LAMBDA_1 * loss_s
    return (loss, loss_r, LAMBDA_1 * loss_s)
